# initial kernel scaffold (unmeasured)
import jax
import jax.numpy as jnp
from jax import lax
from jax.experimental import pallas as pl
from jax.experimental.pallas import tpu as pltpu

N_Y = 4


def kernel(x, dy):
    m_per, d = x.shape
    _, f = dy.shape
    chunk = d // N_Y

    def body(x_ref, dy_ref, out_ref, send_buf, recv_buf, send_sems, recv_sems):
        my_x = lax.axis_index("x")
        my_y = lax.axis_index("y")
        my_z = lax.axis_index("z")
        right = (my_y + 1) % N_Y
        left = (my_y - 1) % N_Y

        barrier_sem = pltpu.get_barrier_semaphore()
        for nbr in (left, right):
            pl.semaphore_signal(
                barrier_sem, inc=1,
                device_id=(my_x, nbr, my_z),
                device_id_type=pl.DeviceIdType.MESH,
            )
        pl.semaphore_wait(barrier_sem, 2)

        def partial(c):
            return lax.dot_general(
                x_ref[:, pl.ds(c * chunk, chunk)],
                dy_ref[...],
                (((0,), (0,)), ((), ())),
                preferred_element_type=jnp.float32,
            )

        for s in range(N_Y - 1):
            c = (my_y - 1 - s) % N_Y
            acc = partial(c)
            if s > 0:
                acc = acc + recv_buf[s - 1]
            send_buf[...] = acc
            rdma = pltpu.make_async_remote_copy(
                src_ref=send_buf,
                dst_ref=recv_buf.at[s],
                send_sem=send_sems.at[s],
                recv_sem=recv_sems.at[s],
                device_id=(my_x, right, my_z),
                device_id_type=pl.DeviceIdType.MESH,
            )
            rdma.start()
            rdma.wait()

        out_ref[...] = partial(my_y) + recv_buf[N_Y - 2]

    return pl.pallas_call(
        body,
        out_shape=jax.ShapeDtypeStruct((chunk, f), jnp.float32),
        in_specs=[
            pl.BlockSpec(memory_space=pltpu.VMEM),
            pl.BlockSpec(memory_space=pltpu.VMEM),
        ],
        out_specs=pl.BlockSpec(memory_space=pltpu.VMEM),
        scratch_shapes=[
            pltpu.VMEM((chunk, f), jnp.float32),
            pltpu.VMEM((N_Y - 1, chunk, f), jnp.float32),
            pltpu.SemaphoreType.DMA((N_Y - 1,)),
            pltpu.SemaphoreType.DMA((N_Y - 1,)),
        ],
        compiler_params=pltpu.CompilerParams(collective_id=0),
    )(x, dy)


# baseline (device time: 167951 ns/iter reference)
import jax
import jax.numpy as jnp
from jax import lax
from jax.experimental import pallas as pl
from jax.experimental.pallas import tpu as pltpu

N_Y = 4


def kernel(x, dy):
    m_per, d = x.shape
    _, f = dy.shape
    chunk = d // N_Y

    def body(x_ref, dy_ref, out_ref, send_buf, recv_buf, send_sems, recv_sems):
        my_x = lax.axis_index("x")
        my_y = lax.axis_index("y")
        my_z = lax.axis_index("z")
        right = (my_y + 1) % N_Y
        left = (my_y - 1) % N_Y

        barrier_sem = pltpu.get_barrier_semaphore()
        for nbr in (left, right):
            pl.semaphore_signal(
                barrier_sem, inc=1,
                device_id=(my_x, nbr, my_z),
                device_id_type=pl.DeviceIdType.MESH,
            )
        pl.semaphore_wait(barrier_sem, 2)

        def partial(c):
            return lax.dot_general(
                x_ref[:, pl.ds(c * chunk, chunk)],
                dy_ref[...],
                (((0,), (0,)), ((), ())),
                preferred_element_type=jnp.float32,
            )

        for s in range(N_Y - 1):
            c = (my_y - 1 - s) % N_Y
            acc = partial(c)
            if s > 0:
                acc = acc + recv_buf[s - 1]
            send_buf[...] = acc
            rdma = pltpu.make_async_remote_copy(
                src_ref=send_buf,
                dst_ref=recv_buf.at[s],
                send_sem=send_sems.at[s],
                recv_sem=recv_sems.at[s],
                device_id=(my_x, right, my_z),
                device_id_type=pl.DeviceIdType.MESH,
            )
            rdma.start()
            rdma.wait()

        out_ref[...] = partial(my_y) + recv_buf[N_Y - 2]

    return pl.pallas_call(
        body,
        out_shape=jax.ShapeDtypeStruct((chunk, f), jnp.float32),
        in_specs=[
            pl.BlockSpec(memory_space=pltpu.VMEM),
            pl.BlockSpec(memory_space=pltpu.VMEM),
        ],
        out_specs=pl.BlockSpec(memory_space=pltpu.VMEM),
        scratch_shapes=[
            pltpu.VMEM((chunk, f), jnp.float32),
            pltpu.VMEM((N_Y - 1, chunk, f), jnp.float32),
            pltpu.SemaphoreType.DMA((N_Y - 1,)),
            pltpu.SemaphoreType.DMA((N_Y - 1,)),
        ],
        compiler_params=pltpu.CompilerParams(
            collective_id=0,
            vmem_limit_bytes=64 * 1024 * 1024,
        ),
    )(x, dy)


# device time: 85319 ns/iter; 1.9685x vs baseline; 1.9685x over previous
import jax
import jax.numpy as jnp
from jax import lax
from jax.experimental import pallas as pl
from jax.experimental.pallas import tpu as pltpu

N_Y = 4
N_X = 2
N_Z = 4
N_REP = N_X * N_Z


def kernel(x, dy):
    m_per, d = x.shape
    _, f = dy.shape
    chunk = d // N_Y
    fsl = f // N_REP

    def body(x_ref, dy_ref, out_ref, send_buf, recv_buf,
             p1_send_sems, p1_recv_sems, p2_send_sems, p2_recv_sems):
        my_x = lax.axis_index("x")
        my_y = lax.axis_index("y")
        my_z = lax.axis_index("z")
        rep = my_x * N_Z + my_z

        barrier_sem = pltpu.get_barrier_semaphore()
        for k in range(1, N_Y):
            q = (my_y + k) % N_Y
            pl.semaphore_signal(
                barrier_sem, inc=1,
                device_id=(my_x, q, my_z),
                device_id_type=pl.DeviceIdType.MESH,
            )
        for k in range(1, N_REP):
            rt = (rep + k) % N_REP
            pl.semaphore_signal(
                barrier_sem, inc=1,
                device_id=(rt // N_Z, my_y, rt % N_Z),
                device_id_type=pl.DeviceIdType.MESH,
            )
        pl.semaphore_wait(barrier_sem, (N_Y - 1) + (N_REP - 1))

        def partial(c):
            return lax.dot_general(
                x_ref[:, pl.ds(c * chunk, chunk)],
                dy_ref[:, pl.ds(rep * fsl, fsl)],
                (((0,), (0,)), ((), ())),
                preferred_element_type=jnp.float32,
            )

        p1_sends = []
        for k in range(1, N_Y):
            q = (my_y + k) % N_Y
            send_buf[k - 1] = partial(q)
            rdma = pltpu.make_async_remote_copy(
                src_ref=send_buf.at[k - 1],
                dst_ref=recv_buf.at[my_y],
                send_sem=p1_send_sems.at[q],
                recv_sem=p1_recv_sems.at[my_y],
                device_id=(my_x, q, my_z),
                device_id_type=pl.DeviceIdType.MESH,
            )
            rdma.start()
            p1_sends.append(rdma)

        acc = partial(my_y)

        for k in range(1, N_Y):
            s = (my_y + k) % N_Y
            recv = pltpu.make_async_remote_copy(
                src_ref=recv_buf.at[s],
                dst_ref=recv_buf.at[s],
                send_sem=p1_send_sems.at[s],
                recv_sem=p1_recv_sems.at[s],
                device_id=(my_x, s, my_z),
                device_id_type=pl.DeviceIdType.MESH,
            )
            recv.wait_recv()
            acc = acc + recv_buf[s]

        out_ref[:, pl.ds(rep * fsl, fsl)] = acc

        for rdma in p1_sends:
            rdma.wait_send()

        p2_sends = []
        for k in range(1, N_REP):
            rt = (rep + k) % N_REP
            rdma = pltpu.make_async_remote_copy(
                src_ref=out_ref.at[:, pl.ds(rep * fsl, fsl)],
                dst_ref=out_ref.at[:, pl.ds(rep * fsl, fsl)],
                send_sem=p2_send_sems.at[rt],
                recv_sem=p2_recv_sems.at[rep],
                device_id=(rt // N_Z, my_y, rt % N_Z),
                device_id_type=pl.DeviceIdType.MESH,
            )
            rdma.start()
            p2_sends.append(rdma)

        for k in range(1, N_REP):
            s = (rep + k) % N_REP
            recv = pltpu.make_async_remote_copy(
                src_ref=out_ref.at[:, pl.ds(s * fsl, fsl)],
                dst_ref=out_ref.at[:, pl.ds(s * fsl, fsl)],
                send_sem=p2_send_sems.at[s],
                recv_sem=p2_recv_sems.at[s],
                device_id=(s // N_Z, my_y, s % N_Z),
                device_id_type=pl.DeviceIdType.MESH,
            )
            recv.wait_recv()

        for rdma in p2_sends:
            rdma.wait_send()

    return pl.pallas_call(
        body,
        out_shape=jax.ShapeDtypeStruct((chunk, f), jnp.float32),
        in_specs=[
            pl.BlockSpec(memory_space=pltpu.VMEM),
            pl.BlockSpec(memory_space=pltpu.VMEM),
        ],
        out_specs=pl.BlockSpec(memory_space=pltpu.VMEM),
        scratch_shapes=[
            pltpu.VMEM((N_Y - 1, chunk, fsl), jnp.float32),
            pltpu.VMEM((N_Y, chunk, fsl), jnp.float32),
            pltpu.SemaphoreType.DMA((N_Y,)),
            pltpu.SemaphoreType.DMA((N_Y,)),
            pltpu.SemaphoreType.DMA((N_REP,)),
            pltpu.SemaphoreType.DMA((N_REP,)),
        ],
        compiler_params=pltpu.CompilerParams(
            collective_id=0,
            vmem_limit_bytes=64 * 1024 * 1024,
        ),
    )(x, dy)


# device time: 74333 ns/iter; 2.2594x vs baseline; 1.1478x over previous
import jax
import jax.numpy as jnp
from jax import lax
from jax.experimental import pallas as pl
from jax.experimental.pallas import tpu as pltpu

N_Y = 4
N_X = 2
N_Z = 4
N_REP = N_X * N_Z
N_H = 2


def kernel(x, dy):
    m_per, d = x.shape
    _, f = dy.shape
    chunk = d // N_Y
    fsl = f // N_REP
    fh = fsl // N_H

    def body(x_ref, dy_ref, out_ref, send_buf, recv_buf,
             p1_send_sems, p1_recv_sems, p2_send_sems, p2_recv_sems):
        my_x = lax.axis_index("x")
        my_y = lax.axis_index("y")
        my_z = lax.axis_index("z")
        rep = my_x * N_Z + my_z

        barrier_sem = pltpu.get_barrier_semaphore()
        for k in range(1, N_Y):
            q = (my_y + k) % N_Y
            pl.semaphore_signal(
                barrier_sem, inc=1,
                device_id=(my_x, q, my_z),
                device_id_type=pl.DeviceIdType.MESH,
            )
        for k in range(1, N_REP):
            rt = (rep + k) % N_REP
            pl.semaphore_signal(
                barrier_sem, inc=1,
                device_id=(rt // N_Z, my_y, rt % N_Z),
                device_id_type=pl.DeviceIdType.MESH,
            )
        pl.semaphore_wait(barrier_sem, (N_Y - 1) + (N_REP - 1))

        def partial(c, h):
            return lax.dot_general(
                x_ref[:, pl.ds(c * chunk, chunk)],
                dy_ref[:, pl.ds(rep * fsl + h * fh, fh)],
                (((0,), (0,)), ((), ())),
                preferred_element_type=jnp.float32,
            )

        pending = []

        def phase1(h):
            sends = []
            for k in range(1, N_Y):
                q = (my_y + k) % N_Y
                send_buf[h, k - 1] = partial(q, h)
                rdma = pltpu.make_async_remote_copy(
                    src_ref=send_buf.at[h, k - 1],
                    dst_ref=recv_buf.at[h, my_y],
                    send_sem=p1_send_sems.at[h, q],
                    recv_sem=p1_recv_sems.at[h, my_y],
                    device_id=(my_x, q, my_z),
                    device_id_type=pl.DeviceIdType.MESH,
                )
                rdma.start()
                sends.append(rdma)
            pending.extend(sends)

            acc = partial(my_y, h)
            for k in range(1, N_Y):
                s = (my_y + k) % N_Y
                recv = pltpu.make_async_remote_copy(
                    src_ref=recv_buf.at[h, s],
                    dst_ref=recv_buf.at[h, s],
                    send_sem=p1_send_sems.at[h, s],
                    recv_sem=p1_recv_sems.at[h, s],
                    device_id=(my_x, s, my_z),
                    device_id_type=pl.DeviceIdType.MESH,
                )
                recv.wait_recv()
                acc = acc + recv_buf[h, s]
            out_ref[:, pl.ds(rep * fsl + h * fh, fh)] = acc

        def phase2_send(h):
            for k in range(1, N_REP):
                rt = (rep + k) % N_REP
                rdma = pltpu.make_async_remote_copy(
                    src_ref=out_ref.at[:, pl.ds(rep * fsl + h * fh, fh)],
                    dst_ref=out_ref.at[:, pl.ds(rep * fsl + h * fh, fh)],
                    send_sem=p2_send_sems.at[h, rt],
                    recv_sem=p2_recv_sems.at[h, rep],
                    device_id=(rt // N_Z, my_y, rt % N_Z),
                    device_id_type=pl.DeviceIdType.MESH,
                )
                rdma.start()
                pending.append(rdma)

        def phase2_wait(h):
            for k in range(1, N_REP):
                s = (rep + k) % N_REP
                recv = pltpu.make_async_remote_copy(
                    src_ref=out_ref.at[:, pl.ds(s * fsl + h * fh, fh)],
                    dst_ref=out_ref.at[:, pl.ds(s * fsl + h * fh, fh)],
                    send_sem=p2_send_sems.at[h, s],
                    recv_sem=p2_recv_sems.at[h, s],
                    device_id=(s // N_Z, my_y, s % N_Z),
                    device_id_type=pl.DeviceIdType.MESH,
                )
                recv.wait_recv()

        for h in range(N_H):
            phase1(h)
            phase2_send(h)
        for h in range(N_H):
            phase2_wait(h)
        for rdma in pending:
            rdma.wait_send()

    return pl.pallas_call(
        body,
        out_shape=jax.ShapeDtypeStruct((chunk, f), jnp.float32),
        in_specs=[
            pl.BlockSpec(memory_space=pltpu.VMEM),
            pl.BlockSpec(memory_space=pltpu.VMEM),
        ],
        out_specs=pl.BlockSpec(memory_space=pltpu.VMEM),
        scratch_shapes=[
            pltpu.VMEM((N_H, N_Y - 1, chunk, fh), jnp.float32),
            pltpu.VMEM((N_H, N_Y, chunk, fh), jnp.float32),
            pltpu.SemaphoreType.DMA((N_H, N_Y)),
            pltpu.SemaphoreType.DMA((N_H, N_Y)),
            pltpu.SemaphoreType.DMA((N_H, N_REP)),
            pltpu.SemaphoreType.DMA((N_H, N_REP)),
        ],
        compiler_params=pltpu.CompilerParams(
            collective_id=0,
            vmem_limit_bytes=64 * 1024 * 1024,
        ),
    )(x, dy)


# device time: 73160 ns/iter; 2.2957x vs baseline; 1.0160x over previous
import jax
import jax.numpy as jnp
from jax import lax
from jax.experimental import pallas as pl
from jax.experimental.pallas import tpu as pltpu

N_Y = 4
N_X = 2
N_Z = 4
N_REP = N_X * N_Z
N_H = 4


def kernel(x, dy):
    m_per, d = x.shape
    _, f = dy.shape
    chunk = d // N_Y
    fsl = f // N_REP
    fh = fsl // N_H

    def body(x_ref, dy_ref, out_ref, send_buf, recv_buf,
             p1_send_sems, p1_recv_sems, p2_send_sems, p2_recv_sems):
        my_x = lax.axis_index("x")
        my_y = lax.axis_index("y")
        my_z = lax.axis_index("z")
        rep = my_x * N_Z + my_z

        barrier_sem = pltpu.get_barrier_semaphore()
        for k in range(1, N_Y):
            q = (my_y + k) % N_Y
            pl.semaphore_signal(
                barrier_sem, inc=1,
                device_id=(my_x, q, my_z),
                device_id_type=pl.DeviceIdType.MESH,
            )
        for k in range(1, N_REP):
            rt = (rep + k) % N_REP
            pl.semaphore_signal(
                barrier_sem, inc=1,
                device_id=(rt // N_Z, my_y, rt % N_Z),
                device_id_type=pl.DeviceIdType.MESH,
            )
        pl.semaphore_wait(barrier_sem, (N_Y - 1) + (N_REP - 1))

        def partial(c, h):
            return lax.dot_general(
                x_ref[:, pl.ds(c * chunk, chunk)],
                dy_ref[:, pl.ds(rep * fsl + h * fh, fh)],
                (((0,), (0,)), ((), ())),
                preferred_element_type=jnp.float32,
            )

        pending = []

        def phase1(h):
            sends = []
            for k in range(1, N_Y):
                q = (my_y + k) % N_Y
                send_buf[h, k - 1] = partial(q, h)
                rdma = pltpu.make_async_remote_copy(
                    src_ref=send_buf.at[h, k - 1],
                    dst_ref=recv_buf.at[h, my_y],
                    send_sem=p1_send_sems.at[h, q],
                    recv_sem=p1_recv_sems.at[h, my_y],
                    device_id=(my_x, q, my_z),
                    device_id_type=pl.DeviceIdType.MESH,
                )
                rdma.start()
                sends.append(rdma)
            pending.extend(sends)

            acc = partial(my_y, h)
            for k in range(1, N_Y):
                s = (my_y + k) % N_Y
                recv = pltpu.make_async_remote_copy(
                    src_ref=recv_buf.at[h, s],
                    dst_ref=recv_buf.at[h, s],
                    send_sem=p1_send_sems.at[h, s],
                    recv_sem=p1_recv_sems.at[h, s],
                    device_id=(my_x, s, my_z),
                    device_id_type=pl.DeviceIdType.MESH,
                )
                recv.wait_recv()
                acc = acc + recv_buf[h, s]
            out_ref[:, pl.ds(rep * fsl + h * fh, fh)] = acc

        def phase2_send(h):
            for k in range(1, N_REP):
                rt = (rep + k) % N_REP
                rdma = pltpu.make_async_remote_copy(
                    src_ref=out_ref.at[:, pl.ds(rep * fsl + h * fh, fh)],
                    dst_ref=out_ref.at[:, pl.ds(rep * fsl + h * fh, fh)],
                    send_sem=p2_send_sems.at[h, rt],
                    recv_sem=p2_recv_sems.at[h, rep],
                    device_id=(rt // N_Z, my_y, rt % N_Z),
                    device_id_type=pl.DeviceIdType.MESH,
                )
                rdma.start()
                pending.append(rdma)

        def phase2_wait(h):
            for k in range(1, N_REP):
                s = (rep + k) % N_REP
                recv = pltpu.make_async_remote_copy(
                    src_ref=out_ref.at[:, pl.ds(s * fsl + h * fh, fh)],
                    dst_ref=out_ref.at[:, pl.ds(s * fsl + h * fh, fh)],
                    send_sem=p2_send_sems.at[h, s],
                    recv_sem=p2_recv_sems.at[h, s],
                    device_id=(s // N_Z, my_y, s % N_Z),
                    device_id_type=pl.DeviceIdType.MESH,
                )
                recv.wait_recv()

        for h in range(N_H):
            phase1(h)
            phase2_send(h)
        for h in range(N_H):
            phase2_wait(h)
        for rdma in pending:
            rdma.wait_send()

    return pl.pallas_call(
        body,
        out_shape=jax.ShapeDtypeStruct((chunk, f), jnp.float32),
        in_specs=[
            pl.BlockSpec(memory_space=pltpu.VMEM),
            pl.BlockSpec(memory_space=pltpu.VMEM),
        ],
        out_specs=pl.BlockSpec(memory_space=pltpu.VMEM),
        scratch_shapes=[
            pltpu.VMEM((N_H, N_Y - 1, chunk, fh), jnp.float32),
            pltpu.VMEM((N_H, N_Y, chunk, fh), jnp.float32),
            pltpu.SemaphoreType.DMA((N_H, N_Y)),
            pltpu.SemaphoreType.DMA((N_H, N_Y)),
            pltpu.SemaphoreType.DMA((N_H, N_REP)),
            pltpu.SemaphoreType.DMA((N_H, N_REP)),
        ],
        compiler_params=pltpu.CompilerParams(
            collective_id=0,
            vmem_limit_bytes=64 * 1024 * 1024,
        ),
    )(x, dy)


# device time: 68633 ns/iter; 2.4471x vs baseline; 1.0660x over previous
import jax
import jax.numpy as jnp
from jax import lax
from jax.experimental import pallas as pl
from jax.experimental.pallas import tpu as pltpu

N_Y = 4
N_X = 2
N_Z = 4
N_REP = N_X * N_Z
N_H = 2


def kernel(x, dy):
    m_per, d = x.shape
    _, f = dy.shape
    chunk = d // N_Y
    fsl = f // N_REP
    fh = fsl // N_H

    def body(x_ref, dy_ref, out_ref, send_buf, recv_buf,
             p1_send_sems, p1_recv_sems,
             z_recv_sems, x_recv_sems, own_send_sems, relay_send_sems):
        my_x = lax.axis_index("x")
        my_y = lax.axis_index("y")
        my_z = lax.axis_index("z")
        rep = my_x * N_Z + my_z
        xp = 1 - my_x
        has_l = my_z > 0
        has_r = my_z < N_Z - 1
        zl = jnp.maximum(my_z - 1, 0)
        zr = jnp.minimum(my_z + 1, N_Z - 1)

        barrier_sem = pltpu.get_barrier_semaphore()
        for k in range(1, N_Y):
            q = (my_y + k) % N_Y
            pl.semaphore_signal(
                barrier_sem, inc=1,
                device_id=(my_x, q, my_z),
                device_id_type=pl.DeviceIdType.MESH,
            )
        pl.semaphore_signal(
            barrier_sem, inc=1,
            device_id=(xp, my_y, my_z),
            device_id_type=pl.DeviceIdType.MESH,
        )

        @pl.when(has_l)
        def _():
            pl.semaphore_signal(
                barrier_sem, inc=1,
                device_id=(my_x, my_y, my_z - 1),
                device_id_type=pl.DeviceIdType.MESH,
            )

        @pl.when(has_r)
        def _():
            pl.semaphore_signal(
                barrier_sem, inc=1,
                device_id=(my_x, my_y, my_z + 1),
                device_id_type=pl.DeviceIdType.MESH,
            )

        n_nbr = (N_Y - 1) + 1 + has_l.astype(jnp.int32) + has_r.astype(jnp.int32)
        pl.semaphore_wait(barrier_sem, n_nbr)

        def partial(c, h):
            return lax.dot_general(
                x_ref[:, pl.ds(c * chunk, chunk)],
                dy_ref[:, pl.ds(rep * fsl + h * fh, fh)],
                (((0,), (0,)), ((), ())),
                preferred_element_type=jnp.float32,
            )

        def piece_ref(j, h):
            return out_ref.at[:, pl.ds((my_x * N_Z + j) * fsl + h * fh, fh)]

        def xline_piece_ref(j, h):
            return out_ref.at[:, pl.ds((xp * N_Z + j) * fsl + h * fh, fh)]

        pending = []
        always_pending = []

        def send(src, dst, send_sem, recv_sem, dev, cond=None):
            rdma = pltpu.make_async_remote_copy(
                src_ref=src, dst_ref=dst, send_sem=send_sem,
                recv_sem=recv_sem, device_id=dev,
                device_id_type=pl.DeviceIdType.MESH,
            )
            if cond is None:
                rdma.start()
                always_pending.append(rdma)
            else:
                @pl.when(cond)
                def _():
                    rdma.start()
                pending.append((cond, rdma))

        def recv_wait(dst, recv_sem, dev, cond=None):
            rdma = pltpu.make_async_remote_copy(
                src_ref=dst, dst_ref=dst, send_sem=recv_sem,
                recv_sem=recv_sem, device_id=dev,
                device_id_type=pl.DeviceIdType.MESH,
            )
            if cond is None:
                rdma.wait_recv()
            else:
                @pl.when(cond)
                def _():
                    rdma.wait_recv()

        def phase1(h):
            for k in range(1, N_Y):
                q = (my_y + k) % N_Y
                send_buf[h, k - 1] = partial(q, h)
                send(
                    send_buf.at[h, k - 1], recv_buf.at[h, my_y],
                    p1_send_sems.at[h, q], p1_recv_sems.at[h, my_y],
                    (my_x, q, my_z),
                )
            acc = partial(my_y, h)
            for k in range(1, N_Y):
                s = (my_y + k) % N_Y
                recv_wait(recv_buf.at[h, s], p1_recv_sems.at[h, s],
                          (my_x, s, my_z))
                acc = acc + recv_buf[h, s]
            out_ref[:, pl.ds(rep * fsl + h * fh, fh)] = acc

        def phase2_send_own(h):
            send(piece_ref(my_z, h), piece_ref(my_z, h),
                 own_send_sems.at[h, 0], z_recv_sems.at[h, my_z],
                 (my_x, my_y, zl), cond=has_l)
            send(piece_ref(my_z, h), piece_ref(my_z, h),
                 own_send_sems.at[h, 1], z_recv_sems.at[h, my_z],
                 (my_x, my_y, zr), cond=has_r)
            send(piece_ref(my_z, h), piece_ref(my_z, h),
                 own_send_sems.at[h, 2], x_recv_sems.at[h, my_z],
                 (xp, my_y, my_z))

        def phase2_relay(h):
            for dd in range(1, N_Z):
                d = jnp.int32(dd)
                fl = my_z >= d
                jl = jnp.maximum(my_z - d, 0)
                recv_wait(piece_ref(jl, h), z_recv_sems.at[h, jl],
                          (my_x, my_y, zl), cond=fl)
                send(piece_ref(jl, h), piece_ref(jl, h),
                     relay_send_sems.at[h, jl, 0], z_recv_sems.at[h, jl],
                     (my_x, my_y, zr), cond=fl & has_r)
                send(piece_ref(jl, h), piece_ref(jl, h),
                     relay_send_sems.at[h, jl, 1], x_recv_sems.at[h, jl],
                     (xp, my_y, my_z), cond=fl)
                fr = my_z + d <= N_Z - 1
                jr = jnp.minimum(my_z + d, N_Z - 1)
                recv_wait(piece_ref(jr, h), z_recv_sems.at[h, jr],
                          (my_x, my_y, zr), cond=fr)
                send(piece_ref(jr, h), piece_ref(jr, h),
                     relay_send_sems.at[h, jr, 0], z_recv_sems.at[h, jr],
                     (my_x, my_y, zl), cond=fr & has_l)
                send(piece_ref(jr, h), piece_ref(jr, h),
                     relay_send_sems.at[h, jr, 1], x_recv_sems.at[h, jr],
                     (xp, my_y, my_z), cond=fr)

        def phase2_xwait(h):
            for j in range(N_Z):
                recv_wait(xline_piece_ref(j, h), x_recv_sems.at[h, j],
                          (xp, my_y, my_z))

        for h in range(N_H):
            phase1(h)
            phase2_send_own(h)
        for h in range(N_H):
            phase2_relay(h)
        for h in range(N_H):
            phase2_xwait(h)
        for cond, rdma in pending:
            @pl.when(cond)
            def _():
                rdma.wait_send()
        for rdma in always_pending:
            rdma.wait_send()

    return pl.pallas_call(
        body,
        out_shape=jax.ShapeDtypeStruct((chunk, f), jnp.float32),
        in_specs=[
            pl.BlockSpec(memory_space=pltpu.VMEM),
            pl.BlockSpec(memory_space=pltpu.VMEM),
        ],
        out_specs=pl.BlockSpec(memory_space=pltpu.VMEM),
        scratch_shapes=[
            pltpu.VMEM((N_H, N_Y - 1, chunk, fh), jnp.float32),
            pltpu.VMEM((N_H, N_Y, chunk, fh), jnp.float32),
            pltpu.SemaphoreType.DMA((N_H, N_Y)),
            pltpu.SemaphoreType.DMA((N_H, N_Y)),
            pltpu.SemaphoreType.DMA((N_H, N_Z)),
            pltpu.SemaphoreType.DMA((N_H, N_Z)),
            pltpu.SemaphoreType.DMA((N_H, 3)),
            pltpu.SemaphoreType.DMA((N_H, N_Z, 2)),
        ],
        compiler_params=pltpu.CompilerParams(
            collective_id=0,
            vmem_limit_bytes=64 * 1024 * 1024,
        ),
    )(x, dy)
